# initial kernel scaffold (unmeasured)
import functools

import jax
import jax.numpy as jnp
from jax import lax
from jax.experimental import pallas as pl
from jax.experimental.pallas import tpu as pltpu

N_DEV = 4
M_CH = 1024
N_OUT = 2048
K_LOC = 1024


def kernel(x, w_mat, scale_x, scale_w):
    def body(x_ref, w_ref, sx_ref, sw_ref, out_ref,
             comm_ref, send_sems, recv_sems):
        my = lax.axis_index("i")
        left = lax.rem(my + (N_DEV - 1), N_DEV)
        right = lax.rem(my + 1, N_DEV)

        barrier_sem = pltpu.get_barrier_semaphore()
        for nbr in (left, right):
            pl.semaphore_signal(
                barrier_sem, inc=1,
                device_id=(nbr,), device_id_type=pl.DeviceIdType.MESH,
            )
        pl.semaphore_wait(barrier_sem, 2)

        def partial(c):
            a = x_ref[pl.ds(c * M_CH, M_CH), :]
            return lax.dot_general(
                a, w_ref[...],
                dimension_numbers=(((1,), (0,)), ((), ())),
                preferred_element_type=jnp.int32,
            )

        comm_ref[3, :, :] = partial(lax.rem(my + (N_DEV - 1), N_DEV))

        for h in range(N_DEV - 1):
            src_slot = 3 if h == 0 else h - 1
            rdma = pltpu.make_async_remote_copy(
                src_ref=comm_ref.at[src_slot],
                dst_ref=comm_ref.at[h],
                send_sem=send_sems.at[h],
                recv_sem=recv_sems.at[h],
                device_id=(right,),
                device_id_type=pl.DeviceIdType.MESH,
            )
            rdma.start()
            rdma.wait()

            rc = lax.rem(my + ((-2 - h) % N_DEV), N_DEV)
            if h < N_DEV - 2:
                comm_ref[h, :, :] = comm_ref[h, :, :] + partial(rc)
            else:
                scale = sx_ref[0] * sw_ref[0]
                acc = comm_ref[h, :, :] + partial(rc)
                out_ref[...] = acc.astype(jnp.float32) * scale

        @functools.partial(
            pl.run_scoped, second_barrier=pltpu.SemaphoreType.REGULAR
        )
        def _(second_barrier):
            for nbr in (left, right):
                pl.semaphore_signal(
                    second_barrier, inc=1,
                    device_id=(nbr,), device_id_type=pl.DeviceIdType.MESH,
                )
            pl.semaphore_wait(second_barrier, 2)

    return pl.pallas_call(
        body,
        out_shape=jax.ShapeDtypeStruct((M_CH, N_OUT), jnp.float32),
        in_specs=[
            pl.BlockSpec(memory_space=pltpu.VMEM),
            pl.BlockSpec(memory_space=pltpu.VMEM),
            pl.BlockSpec(memory_space=pltpu.SMEM),
            pl.BlockSpec(memory_space=pltpu.SMEM),
        ],
        out_specs=pl.BlockSpec(memory_space=pltpu.VMEM),
        scratch_shapes=[
            pltpu.VMEM((N_DEV, M_CH, N_OUT), jnp.int32),
            pltpu.SemaphoreType.DMA((N_DEV - 1,)),
            pltpu.SemaphoreType.DMA((N_DEV - 1,)),
        ],
        compiler_params=pltpu.CompilerParams(collective_id=0),
    )(x, w_mat, scale_x, scale_w)


# baseline (device time: 310579 ns/iter reference)
import functools

import jax
import jax.numpy as jnp
from jax import lax
from jax.experimental import pallas as pl
from jax.experimental.pallas import tpu as pltpu

N_DEV = 4
M_CH = 1024
N_OUT = 2048
K_LOC = 1024


def kernel(x, w_mat, scale_x, scale_w):
    def body(x_ref, w_ref, sx_ref, sw_ref, out_ref,
             comm_ref, send_sems, recv_sems):
        my = lax.axis_index("i")
        left = lax.rem(my + (N_DEV - 1), N_DEV)
        right = lax.rem(my + 1, N_DEV)

        barrier_sem = pltpu.get_barrier_semaphore()
        for nbr in (left, right):
            pl.semaphore_signal(
                barrier_sem, inc=1,
                device_id=(nbr,), device_id_type=pl.DeviceIdType.MESH,
            )
        pl.semaphore_wait(barrier_sem, 2)

        def partial(c):
            a = x_ref[pl.ds(c * M_CH, M_CH), :]
            return lax.dot_general(
                a, w_ref[...],
                dimension_numbers=(((1,), (0,)), ((), ())),
                preferred_element_type=jnp.int32,
            )

        comm_ref[3, :, :] = partial(lax.rem(my + (N_DEV - 1), N_DEV))

        for h in range(N_DEV - 1):
            src_slot = 3 if h == 0 else h - 1
            rdma = pltpu.make_async_remote_copy(
                src_ref=comm_ref.at[src_slot],
                dst_ref=comm_ref.at[h],
                send_sem=send_sems.at[h],
                recv_sem=recv_sems.at[h],
                device_id=(right,),
                device_id_type=pl.DeviceIdType.MESH,
            )
            rdma.start()
            rdma.wait()

            rc = lax.rem(my + ((-2 - h) % N_DEV), N_DEV)
            if h < N_DEV - 2:
                comm_ref[h, :, :] = comm_ref[h, :, :] + partial(rc)
            else:
                scale = sx_ref[0] * sw_ref[0]
                acc = comm_ref[h, :, :] + partial(rc)
                out_ref[...] = acc.astype(jnp.float32) * scale

        @functools.partial(
            pl.run_scoped, second_barrier=pltpu.SemaphoreType.REGULAR
        )
        def _(second_barrier):
            for nbr in (left, right):
                pl.semaphore_signal(
                    second_barrier, inc=1,
                    device_id=(nbr,), device_id_type=pl.DeviceIdType.MESH,
                )
            pl.semaphore_wait(second_barrier, 2)

    return pl.pallas_call(
        body,
        out_shape=jax.ShapeDtypeStruct((M_CH, N_OUT), jnp.float32),
        in_specs=[
            pl.BlockSpec(memory_space=pltpu.VMEM),
            pl.BlockSpec(memory_space=pltpu.VMEM),
            pl.BlockSpec(memory_space=pltpu.SMEM),
            pl.BlockSpec(memory_space=pltpu.SMEM),
        ],
        out_specs=pl.BlockSpec(memory_space=pltpu.VMEM),
        scratch_shapes=[
            pltpu.VMEM((N_DEV, M_CH, N_OUT), jnp.int32),
            pltpu.SemaphoreType.DMA((N_DEV - 1,)),
            pltpu.SemaphoreType.DMA((N_DEV - 1,)),
        ],
        compiler_params=pltpu.CompilerParams(
            collective_id=0,
            vmem_limit_bytes=100 * 1024 * 1024,
        ),
    )(x, w_mat, scale_x, scale_w)


# device time: 175795 ns/iter; 1.7667x vs baseline; 1.7667x over previous
import functools

import jax
import jax.numpy as jnp
from jax import lax
from jax.experimental import pallas as pl
from jax.experimental.pallas import tpu as pltpu

N_DEV = 4
M_CH = 1024
N_OUT = 2048
N_HALF = N_OUT // 2


def kernel(x, w_mat, scale_x, scale_w):
    def body(x_ref, w_ref, sx_ref, sw_ref, out_ref,
             cw_ref, ccw_ref,
             send_cw, recv_cw, send_ccw, recv_ccw):
        my = lax.axis_index("i")
        left = lax.rem(my + (N_DEV - 1), N_DEV)
        right = lax.rem(my + 1, N_DEV)

        barrier_sem = pltpu.get_barrier_semaphore()
        for nbr in (left, right):
            pl.semaphore_signal(
                barrier_sem, inc=1,
                device_id=(nbr,), device_id_type=pl.DeviceIdType.MESH,
            )
        pl.semaphore_wait(barrier_sem, 2)

        def partial(c, half):
            a = x_ref[pl.ds(c * M_CH, M_CH), :]
            b = w_ref[:, pl.ds(half * N_HALF, N_HALF)]
            return lax.dot_general(
                a, b,
                dimension_numbers=(((1,), (0,)), ((), ())),
                preferred_element_type=jnp.int32,
            )

        cw_ref[3, :, :] = partial(lax.rem(my + (N_DEV - 1), N_DEV), 0)
        ccw_ref[3, :, :] = partial(lax.rem(my + 1, N_DEV), 1)

        for h in range(N_DEV - 1):
            src_slot = 3 if h == 0 else h - 1
            rdma_cw = pltpu.make_async_remote_copy(
                src_ref=cw_ref.at[src_slot],
                dst_ref=cw_ref.at[h],
                send_sem=send_cw.at[h],
                recv_sem=recv_cw.at[h],
                device_id=(right,),
                device_id_type=pl.DeviceIdType.MESH,
            )
            rdma_ccw = pltpu.make_async_remote_copy(
                src_ref=ccw_ref.at[src_slot],
                dst_ref=ccw_ref.at[h],
                send_sem=send_ccw.at[h],
                recv_sem=recv_ccw.at[h],
                device_id=(left,),
                device_id_type=pl.DeviceIdType.MESH,
            )
            rdma_cw.start()
            rdma_ccw.start()
            rdma_cw.wait()
            rdma_ccw.wait()

            rc_cw = lax.rem(my + ((-2 - h) % N_DEV), N_DEV)
            rc_ccw = lax.rem(my + ((2 + h) % N_DEV), N_DEV)
            if h < N_DEV - 2:
                cw_ref[h, :, :] = cw_ref[h, :, :] + partial(rc_cw, 0)
                ccw_ref[h, :, :] = ccw_ref[h, :, :] + partial(rc_ccw, 1)
            else:
                scale = sx_ref[0] * sw_ref[0]
                acc_l = cw_ref[h, :, :] + partial(rc_cw, 0)
                acc_r = ccw_ref[h, :, :] + partial(rc_ccw, 1)
                out_ref[:, pl.ds(0, N_HALF)] = (
                    acc_l.astype(jnp.float32) * scale
                )
                out_ref[:, pl.ds(N_HALF, N_HALF)] = (
                    acc_r.astype(jnp.float32) * scale
                )

        @functools.partial(
            pl.run_scoped, second_barrier=pltpu.SemaphoreType.REGULAR
        )
        def _(second_barrier):
            for nbr in (left, right):
                pl.semaphore_signal(
                    second_barrier, inc=1,
                    device_id=(nbr,), device_id_type=pl.DeviceIdType.MESH,
                )
            pl.semaphore_wait(second_barrier, 2)

    return pl.pallas_call(
        body,
        out_shape=jax.ShapeDtypeStruct((M_CH, N_OUT), jnp.float32),
        in_specs=[
            pl.BlockSpec(memory_space=pltpu.VMEM),
            pl.BlockSpec(memory_space=pltpu.VMEM),
            pl.BlockSpec(memory_space=pltpu.SMEM),
            pl.BlockSpec(memory_space=pltpu.SMEM),
        ],
        out_specs=pl.BlockSpec(memory_space=pltpu.VMEM),
        scratch_shapes=[
            pltpu.VMEM((N_DEV, M_CH, N_HALF), jnp.int32),
            pltpu.VMEM((N_DEV, M_CH, N_HALF), jnp.int32),
            pltpu.SemaphoreType.DMA((N_DEV - 1,)),
            pltpu.SemaphoreType.DMA((N_DEV - 1,)),
            pltpu.SemaphoreType.DMA((N_DEV - 1,)),
            pltpu.SemaphoreType.DMA((N_DEV - 1,)),
        ],
        compiler_params=pltpu.CompilerParams(
            collective_id=0,
            vmem_limit_bytes=100 * 1024 * 1024,
        ),
    )(x, w_mat, scale_x, scale_w)


# device time: 163439 ns/iter; 1.9003x vs baseline; 1.0756x over previous
import functools

import jax
import jax.numpy as jnp
from jax import lax
from jax.experimental import pallas as pl
from jax.experimental.pallas import tpu as pltpu

N_DEV = 4
M_CH = 1024
N_OUT = 2048
N_HALF = N_OUT // 2


def kernel(x, w_mat, scale_x, scale_w):
    def body(x_ref, w_ref, sx_ref, sw_ref, out_ref,
             cw_ref, ccw_ref,
             send_cw, recv_cw, send_ccw, recv_ccw):
        my = lax.axis_index("i")
        left = lax.rem(my + (N_DEV - 1), N_DEV)
        right = lax.rem(my + 1, N_DEV)

        barrier_sem = pltpu.get_barrier_semaphore()
        for nbr in (left, right):
            pl.semaphore_signal(
                barrier_sem, inc=1,
                device_id=(nbr,), device_id_type=pl.DeviceIdType.MESH,
            )
        pl.semaphore_wait(barrier_sem, 2)

        def partial(c, half):
            a = x_ref[pl.ds(c * M_CH, M_CH), :]
            b = w_ref[:, pl.ds(half * N_HALF, N_HALF)]
            return lax.dot_general(
                a, b,
                dimension_numbers=(((1,), (0,)), ((), ())),
                preferred_element_type=jnp.int32,
            )

        def rdma(ring_ref, h, ssems, rsems, tgt):
            src_slot = 3 if h == 0 else h - 1
            return pltpu.make_async_remote_copy(
                src_ref=ring_ref.at[src_slot],
                dst_ref=ring_ref.at[h],
                send_sem=ssems.at[h],
                recv_sem=rsems.at[h],
                device_id=(tgt,),
                device_id_type=pl.DeviceIdType.MESH,
            )

        def rc_cw(h):
            return lax.rem(my + ((-2 - h) % N_DEV), N_DEV)

        def rc_ccw(h):
            return lax.rem(my + ((2 + h) % N_DEV), N_DEV)

        cw_ref[3, :, :] = partial(lax.rem(my + (N_DEV - 1), N_DEV), 0)
        ccw_ref[3, :, :] = partial(lax.rem(my + 1, N_DEV), 1)

        for h in range(N_DEV - 1):
            r_cw = rdma(cw_ref, h, send_cw, recv_cw, right)
            r_ccw = rdma(ccw_ref, h, send_ccw, recv_ccw, left)
            r_cw.start()
            r_ccw.start()
            a_cw = partial(rc_cw(h), 0)
            a_ccw = partial(rc_ccw(h), 1)
            r_cw.wait()
            r_ccw.wait()
            if h < N_DEV - 2:
                cw_ref[h, :, :] = cw_ref[h, :, :] + a_cw
                ccw_ref[h, :, :] = ccw_ref[h, :, :] + a_ccw
            else:
                scale = sx_ref[0] * sw_ref[0]
                acc_l = cw_ref[h, :, :] + a_cw
                acc_r = ccw_ref[h, :, :] + a_ccw
                out_ref[:, pl.ds(0, N_HALF)] = (
                    acc_l.astype(jnp.float32) * scale
                )
                out_ref[:, pl.ds(N_HALF, N_HALF)] = (
                    acc_r.astype(jnp.float32) * scale
                )

        @functools.partial(
            pl.run_scoped, second_barrier=pltpu.SemaphoreType.REGULAR
        )
        def _(second_barrier):
            for nbr in (left, right):
                pl.semaphore_signal(
                    second_barrier, inc=1,
                    device_id=(nbr,), device_id_type=pl.DeviceIdType.MESH,
                )
            pl.semaphore_wait(second_barrier, 2)

    return pl.pallas_call(
        body,
        out_shape=jax.ShapeDtypeStruct((M_CH, N_OUT), jnp.float32),
        in_specs=[
            pl.BlockSpec(memory_space=pltpu.VMEM),
            pl.BlockSpec(memory_space=pltpu.VMEM),
            pl.BlockSpec(memory_space=pltpu.SMEM),
            pl.BlockSpec(memory_space=pltpu.SMEM),
        ],
        out_specs=pl.BlockSpec(memory_space=pltpu.VMEM),
        scratch_shapes=[
            pltpu.VMEM((N_DEV, M_CH, N_HALF), jnp.int32),
            pltpu.VMEM((N_DEV, M_CH, N_HALF), jnp.int32),
            pltpu.SemaphoreType.DMA((N_DEV - 1,)),
            pltpu.SemaphoreType.DMA((N_DEV - 1,)),
            pltpu.SemaphoreType.DMA((N_DEV - 1,)),
            pltpu.SemaphoreType.DMA((N_DEV - 1,)),
        ],
        compiler_params=pltpu.CompilerParams(
            collective_id=0,
            vmem_limit_bytes=100 * 1024 * 1024,
        ),
    )(x, w_mat, scale_x, scale_w)


# device time: 97750 ns/iter; 3.1773x vs baseline; 1.6720x over previous
import functools

import jax
import jax.numpy as jnp
from jax import lax
from jax.experimental import pallas as pl
from jax.experimental.pallas import tpu as pltpu

N_DEV = 4
M_CH = 1024
N_OUT = 2048
N_HALF = N_OUT // 2


def kernel(x, w_mat, scale_x, scale_w):
    def body(x_ref, w_ref, sx_ref, sw_ref, out_ref,
             cw_ref, ccw_ref,
             send_cw, recv_cw, send_ccw, recv_ccw):
        my = lax.axis_index("i")
        left = lax.rem(my + (N_DEV - 1), N_DEV)
        right = lax.rem(my + 1, N_DEV)

        barrier_sem = pltpu.get_barrier_semaphore()
        for nbr in (left, right):
            pl.semaphore_signal(
                barrier_sem, inc=1,
                device_id=(nbr,), device_id_type=pl.DeviceIdType.MESH,
            )
        pl.semaphore_wait(barrier_sem, 2)

        def partial(c, half):
            a = x_ref[pl.ds(c * M_CH, M_CH), :]
            b = w_ref[:, pl.ds(half * N_HALF, N_HALF)]
            return lax.dot_general(
                a, b,
                dimension_numbers=(((1,), (0,)), ((), ())),
                preferred_element_type=jnp.int32,
            )

        def rdma(ring_ref, h, ssems, rsems, tgt):
            src_slot = 3 if h == 0 else h - 1
            return pltpu.make_async_remote_copy(
                src_ref=ring_ref.at[src_slot],
                dst_ref=ring_ref.at[h],
                send_sem=ssems.at[h],
                recv_sem=rsems.at[h],
                device_id=(tgt,),
                device_id_type=pl.DeviceIdType.MESH,
            )

        def rc_cw(h):
            return lax.rem(my + ((-2 - h) % N_DEV), N_DEV)

        def rc_ccw(h):
            return lax.rem(my + ((2 + h) % N_DEV), N_DEV)

        cw_ref[3, :, :] = partial(
            lax.rem(my + (N_DEV - 1), N_DEV), 0
        ).astype(jnp.bfloat16)
        ccw_ref[3, :, :] = partial(
            lax.rem(my + 1, N_DEV), 1
        ).astype(jnp.bfloat16)

        for h in range(N_DEV - 1):
            r_cw = rdma(cw_ref, h, send_cw, recv_cw, right)
            r_ccw = rdma(ccw_ref, h, send_ccw, recv_ccw, left)
            r_cw.start()
            r_ccw.start()
            a_cw = partial(rc_cw(h), 0)
            a_ccw = partial(rc_ccw(h), 1)
            r_cw.wait()
            r_ccw.wait()
            if h < N_DEV - 2:
                cw_ref[h, :, :] = (
                    cw_ref[h, :, :].astype(jnp.float32)
                    + a_cw.astype(jnp.float32)
                ).astype(jnp.bfloat16)
                ccw_ref[h, :, :] = (
                    ccw_ref[h, :, :].astype(jnp.float32)
                    + a_ccw.astype(jnp.float32)
                ).astype(jnp.bfloat16)
            else:
                scale = sx_ref[0] * sw_ref[0]
                acc_l = (cw_ref[h, :, :].astype(jnp.float32)
                         + a_cw.astype(jnp.float32))
                acc_r = (ccw_ref[h, :, :].astype(jnp.float32)
                         + a_ccw.astype(jnp.float32))
                out_ref[:, pl.ds(0, N_HALF)] = (
                    acc_l.astype(jnp.float32) * scale
                )
                out_ref[:, pl.ds(N_HALF, N_HALF)] = (
                    acc_r.astype(jnp.float32) * scale
                )

        @functools.partial(
            pl.run_scoped, second_barrier=pltpu.SemaphoreType.REGULAR
        )
        def _(second_barrier):
            for nbr in (left, right):
                pl.semaphore_signal(
                    second_barrier, inc=1,
                    device_id=(nbr,), device_id_type=pl.DeviceIdType.MESH,
                )
            pl.semaphore_wait(second_barrier, 2)

    return pl.pallas_call(
        body,
        out_shape=jax.ShapeDtypeStruct((M_CH, N_OUT), jnp.float32),
        in_specs=[
            pl.BlockSpec(memory_space=pltpu.VMEM),
            pl.BlockSpec(memory_space=pltpu.VMEM),
            pl.BlockSpec(memory_space=pltpu.SMEM),
            pl.BlockSpec(memory_space=pltpu.SMEM),
        ],
        out_specs=pl.BlockSpec(memory_space=pltpu.VMEM),
        scratch_shapes=[
            pltpu.VMEM((N_DEV, M_CH, N_HALF), jnp.bfloat16),
            pltpu.VMEM((N_DEV, M_CH, N_HALF), jnp.bfloat16),
            pltpu.SemaphoreType.DMA((N_DEV - 1,)),
            pltpu.SemaphoreType.DMA((N_DEV - 1,)),
            pltpu.SemaphoreType.DMA((N_DEV - 1,)),
            pltpu.SemaphoreType.DMA((N_DEV - 1,)),
        ],
        compiler_params=pltpu.CompilerParams(
            collective_id=0,
            vmem_limit_bytes=100 * 1024 * 1024,
        ),
    )(x, w_mat, scale_x, scale_w)


# device time: 88417 ns/iter; 3.5127x vs baseline; 1.1056x over previous
import functools

import jax
import jax.numpy as jnp
from jax import lax
from jax.experimental import pallas as pl
from jax.experimental.pallas import tpu as pltpu

N_DEV = 4
M_CH = 1024
N_OUT = 2048
N_HALF = N_OUT // 2
SUB = M_CH // 2


def kernel(x, w_mat, scale_x, scale_w):
    def body(x_ref, w_ref, sx_ref, sw_ref, out_ref,
             cw_ref, ccw_ref,
             send_cw, recv_cw, send_ccw, recv_ccw):
        my = lax.axis_index("i")
        left = lax.rem(my + (N_DEV - 1), N_DEV)
        right = lax.rem(my + 1, N_DEV)

        barrier_sem = pltpu.get_barrier_semaphore()
        for nbr in (left, right):
            pl.semaphore_signal(
                barrier_sem, inc=1,
                device_id=(nbr,), device_id_type=pl.DeviceIdType.MESH,
            )
        pl.semaphore_wait(barrier_sem, 2)

        def partial_sub(c, half, s):
            a = x_ref[pl.ds(c * M_CH + s * SUB, SUB), :]
            b = w_ref[:, pl.ds(half * N_HALF, N_HALF)]
            return lax.dot_general(
                a, b,
                dimension_numbers=(((1,), (0,)), ((), ())),
                preferred_element_type=jnp.int32,
            )

        def rdma_sub(ring_ref, h, s, ssems, rsems, tgt):
            src_slot = 3 if h == 0 else h - 1
            return pltpu.make_async_remote_copy(
                src_ref=ring_ref.at[src_slot, pl.ds(s * SUB, SUB), :],
                dst_ref=ring_ref.at[h, pl.ds(s * SUB, SUB), :],
                send_sem=ssems.at[h, s],
                recv_sem=rsems.at[h, s],
                device_id=(tgt,),
                device_id_type=pl.DeviceIdType.MESH,
            )

        def start_sub(h, s):
            r_cw = rdma_sub(cw_ref, h, s, send_cw, recv_cw, right)
            r_ccw = rdma_sub(ccw_ref, h, s, send_ccw, recv_ccw, left)
            r_cw.start()
            r_ccw.start()
            return r_cw, r_ccw

        def rc_cw(h):
            return lax.rem(my + ((-2 - h) % N_DEV), N_DEV)

        def rc_ccw(h):
            return lax.rem(my + ((2 + h) % N_DEV), N_DEV)

        c0_cw = lax.rem(my + (N_DEV - 1), N_DEV)
        c0_ccw = lax.rem(my + 1, N_DEV)
        cw_ref[3, pl.ds(0, SUB), :] = (
            partial_sub(c0_cw, 0, 0).astype(jnp.bfloat16))
        ccw_ref[3, pl.ds(0, SUB), :] = (
            partial_sub(c0_ccw, 1, 0).astype(jnp.bfloat16))
        fl_cw0, fl_ccw0 = start_sub(0, 0)
        cw_ref[3, pl.ds(SUB, SUB), :] = (
            partial_sub(c0_cw, 0, 1).astype(jnp.bfloat16))
        ccw_ref[3, pl.ds(SUB, SUB), :] = (
            partial_sub(c0_ccw, 1, 1).astype(jnp.bfloat16))
        fl_cw1, fl_ccw1 = start_sub(0, 1)

        scale = sx_ref[0] * sw_ref[0]

        for h in range(N_DEV - 1):
            a_cw = [partial_sub(rc_cw(h), 0, s) for s in range(2)]
            a_ccw = [partial_sub(rc_ccw(h), 1, s) for s in range(2)]

            nxt = [None, None]
            for s in range(2):
                fl_cw, fl_ccw = (fl_cw0, fl_ccw0) if s == 0 else (fl_cw1, fl_ccw1)
                fl_cw.wait()
                fl_ccw.wait()
                rows = pl.ds(s * SUB, SUB)
                if h < N_DEV - 2:
                    cw_ref[h, rows, :] = (
                        cw_ref[h, rows, :].astype(jnp.float32)
                        + a_cw[s].astype(jnp.float32)
                    ).astype(jnp.bfloat16)
                    ccw_ref[h, rows, :] = (
                        ccw_ref[h, rows, :].astype(jnp.float32)
                        + a_ccw[s].astype(jnp.float32)
                    ).astype(jnp.bfloat16)
                    nxt[s] = start_sub(h + 1, s)
                else:
                    acc_l = (cw_ref[h, rows, :].astype(jnp.float32)
                             + a_cw[s].astype(jnp.float32))
                    acc_r = (ccw_ref[h, rows, :].astype(jnp.float32)
                             + a_ccw[s].astype(jnp.float32))
                    out_ref[rows, pl.ds(0, N_HALF)] = acc_l * scale
                    out_ref[rows, pl.ds(N_HALF, N_HALF)] = acc_r * scale

            if h < N_DEV - 2:
                fl_cw0, fl_ccw0 = nxt[0]
                fl_cw1, fl_ccw1 = nxt[1]

        @functools.partial(
            pl.run_scoped, second_barrier=pltpu.SemaphoreType.REGULAR
        )
        def _(second_barrier):
            for nbr in (left, right):
                pl.semaphore_signal(
                    second_barrier, inc=1,
                    device_id=(nbr,), device_id_type=pl.DeviceIdType.MESH,
                )
            pl.semaphore_wait(second_barrier, 2)

    return pl.pallas_call(
        body,
        out_shape=jax.ShapeDtypeStruct((M_CH, N_OUT), jnp.float32),
        in_specs=[
            pl.BlockSpec(memory_space=pltpu.VMEM),
            pl.BlockSpec(memory_space=pltpu.VMEM),
            pl.BlockSpec(memory_space=pltpu.SMEM),
            pl.BlockSpec(memory_space=pltpu.SMEM),
        ],
        out_specs=pl.BlockSpec(memory_space=pltpu.VMEM),
        scratch_shapes=[
            pltpu.VMEM((N_DEV, M_CH, N_HALF), jnp.bfloat16),
            pltpu.VMEM((N_DEV, M_CH, N_HALF), jnp.bfloat16),
            pltpu.SemaphoreType.DMA((N_DEV - 1, 2)),
            pltpu.SemaphoreType.DMA((N_DEV - 1, 2)),
            pltpu.SemaphoreType.DMA((N_DEV - 1, 2)),
            pltpu.SemaphoreType.DMA((N_DEV - 1, 2)),
        ],
        compiler_params=pltpu.CompilerParams(
            collective_id=0,
            vmem_limit_bytes=100 * 1024 * 1024,
        ),
    )(x, w_mat, scale_x, scale_w)


# device time: 87205 ns/iter; 3.5615x vs baseline; 1.0139x over previous
import functools

import jax
import jax.numpy as jnp
from jax import lax
from jax.experimental import pallas as pl
from jax.experimental.pallas import tpu as pltpu

N_DEV = 4
M_CH = 1024
N_OUT = 2048
N_HALF = N_OUT // 2
N_SUB = 4
SUB = M_CH // N_SUB


def kernel(x, w_mat, scale_x, scale_w):
    def body(x_ref, w_ref, sx_ref, sw_ref, out_ref,
             cw_ref, ccw_ref,
             send_cw, recv_cw, send_ccw, recv_ccw):
        my = lax.axis_index("i")
        left = lax.rem(my + (N_DEV - 1), N_DEV)
        right = lax.rem(my + 1, N_DEV)

        barrier_sem = pltpu.get_barrier_semaphore()
        for nbr in (left, right):
            pl.semaphore_signal(
                barrier_sem, inc=1,
                device_id=(nbr,), device_id_type=pl.DeviceIdType.MESH,
            )
        pl.semaphore_wait(barrier_sem, 2)

        def partial_sub(c, half, s):
            a = x_ref[pl.ds(c * M_CH + s * SUB, SUB), :]
            b = w_ref[:, pl.ds(half * N_HALF, N_HALF)]
            return lax.dot_general(
                a, b,
                dimension_numbers=(((1,), (0,)), ((), ())),
                preferred_element_type=jnp.int32,
            )

        def rdma_sub(ring_ref, h, s, ssems, rsems, tgt):
            src_slot = 3 if h == 0 else h - 1
            return pltpu.make_async_remote_copy(
                src_ref=ring_ref.at[src_slot, pl.ds(s * SUB, SUB), :],
                dst_ref=ring_ref.at[h, pl.ds(s * SUB, SUB), :],
                send_sem=ssems.at[h, s],
                recv_sem=rsems.at[h, s],
                device_id=(tgt,),
                device_id_type=pl.DeviceIdType.MESH,
            )

        def start_sub(h, s):
            r_cw = rdma_sub(cw_ref, h, s, send_cw, recv_cw, right)
            r_ccw = rdma_sub(ccw_ref, h, s, send_ccw, recv_ccw, left)
            r_cw.start()
            r_ccw.start()
            return r_cw, r_ccw

        def rc_cw(h):
            return lax.rem(my + ((-2 - h) % N_DEV), N_DEV)

        def rc_ccw(h):
            return lax.rem(my + ((2 + h) % N_DEV), N_DEV)

        c0_cw = lax.rem(my + (N_DEV - 1), N_DEV)
        c0_ccw = lax.rem(my + 1, N_DEV)
        fl = []
        for s in range(N_SUB):
            rows = pl.ds(s * SUB, SUB)
            cw_ref[3, rows, :] = (
                partial_sub(c0_cw, 0, s).astype(jnp.bfloat16))
            ccw_ref[3, rows, :] = (
                partial_sub(c0_ccw, 1, s).astype(jnp.bfloat16))
            fl.append(start_sub(0, s))

        scale = sx_ref[0] * sw_ref[0]

        for h in range(N_DEV - 1):
            a_cw = [partial_sub(rc_cw(h), 0, s) for s in range(N_SUB)]
            a_ccw = [partial_sub(rc_ccw(h), 1, s) for s in range(N_SUB)]

            nxt = []
            for s in range(N_SUB):
                r_cw, r_ccw = fl[s]
                r_cw.wait()
                r_ccw.wait()
                rows = pl.ds(s * SUB, SUB)
                if h < N_DEV - 2:
                    cw_ref[h, rows, :] = (
                        cw_ref[h, rows, :].astype(jnp.float32)
                        + a_cw[s].astype(jnp.float32)
                    ).astype(jnp.bfloat16)
                    ccw_ref[h, rows, :] = (
                        ccw_ref[h, rows, :].astype(jnp.float32)
                        + a_ccw[s].astype(jnp.float32)
                    ).astype(jnp.bfloat16)
                    nxt.append(start_sub(h + 1, s))
                else:
                    acc_l = (cw_ref[h, rows, :].astype(jnp.float32)
                             + a_cw[s].astype(jnp.float32))
                    acc_r = (ccw_ref[h, rows, :].astype(jnp.float32)
                             + a_ccw[s].astype(jnp.float32))
                    out_ref[rows, pl.ds(0, N_HALF)] = acc_l * scale
                    out_ref[rows, pl.ds(N_HALF, N_HALF)] = acc_r * scale

            fl = nxt

        @functools.partial(
            pl.run_scoped, second_barrier=pltpu.SemaphoreType.REGULAR
        )
        def _(second_barrier):
            for nbr in (left, right):
                pl.semaphore_signal(
                    second_barrier, inc=1,
                    device_id=(nbr,), device_id_type=pl.DeviceIdType.MESH,
                )
            pl.semaphore_wait(second_barrier, 2)

    return pl.pallas_call(
        body,
        out_shape=jax.ShapeDtypeStruct((M_CH, N_OUT), jnp.float32),
        in_specs=[
            pl.BlockSpec(memory_space=pltpu.VMEM),
            pl.BlockSpec(memory_space=pltpu.VMEM),
            pl.BlockSpec(memory_space=pltpu.SMEM),
            pl.BlockSpec(memory_space=pltpu.SMEM),
        ],
        out_specs=pl.BlockSpec(memory_space=pltpu.VMEM),
        scratch_shapes=[
            pltpu.VMEM((N_DEV, M_CH, N_HALF), jnp.bfloat16),
            pltpu.VMEM((N_DEV, M_CH, N_HALF), jnp.bfloat16),
            pltpu.SemaphoreType.DMA((N_DEV - 1, N_SUB)),
            pltpu.SemaphoreType.DMA((N_DEV - 1, N_SUB)),
            pltpu.SemaphoreType.DMA((N_DEV - 1, N_SUB)),
            pltpu.SemaphoreType.DMA((N_DEV - 1, N_SUB)),
        ],
        compiler_params=pltpu.CompilerParams(
            collective_id=0,
            vmem_limit_bytes=100 * 1024 * 1024,
        ),
    )(x, w_mat, scale_x, scale_w)


# device time: 87140 ns/iter; 3.5641x vs baseline; 1.0007x over previous
import functools

import jax
import jax.numpy as jnp
from jax import lax
from jax.experimental import pallas as pl
from jax.experimental.pallas import tpu as pltpu

N_DEV = 4
M_CH = 1024
N_OUT = 2048
N_HALF = N_OUT // 2
N_SUB = 4
SUB = M_CH // N_SUB


def kernel(x, w_mat, scale_x, scale_w):
    def body(x_ref, w_ref, sx_ref, sw_ref, out_ref,
             cw_ref, ccw_ref,
             send_cw, recv_cw, send_ccw, recv_ccw):
        my = lax.axis_index("i")
        left = lax.rem(my + (N_DEV - 1), N_DEV)
        right = lax.rem(my + 1, N_DEV)

        def partial_sub(c, half, s):
            a = x_ref[pl.ds(c * M_CH + s * SUB, SUB), :]
            b = w_ref[:, pl.ds(half * N_HALF, N_HALF)]
            return lax.dot_general(
                a, b,
                dimension_numbers=(((1,), (0,)), ((), ())),
                preferred_element_type=jnp.int32,
            )

        def rdma_sub(ring_ref, h, s, ssems, rsems, tgt):
            src_slot = 3 if h == 0 else h - 1
            return pltpu.make_async_remote_copy(
                src_ref=ring_ref.at[src_slot, pl.ds(s * SUB, SUB), :],
                dst_ref=ring_ref.at[h, pl.ds(s * SUB, SUB), :],
                send_sem=ssems.at[h, s],
                recv_sem=rsems.at[h, s],
                device_id=(tgt,),
                device_id_type=pl.DeviceIdType.MESH,
            )

        def start_sub(h, s):
            r_cw = rdma_sub(cw_ref, h, s, send_cw, recv_cw, right)
            r_ccw = rdma_sub(ccw_ref, h, s, send_ccw, recv_ccw, left)
            r_cw.start()
            r_ccw.start()
            return r_cw, r_ccw

        def rc_cw(h):
            return lax.rem(my + ((-2 - h) % N_DEV), N_DEV)

        def rc_ccw(h):
            return lax.rem(my + ((2 + h) % N_DEV), N_DEV)

        c0_cw = lax.rem(my + (N_DEV - 1), N_DEV)
        c0_ccw = lax.rem(my + 1, N_DEV)

        rows0 = pl.ds(0, SUB)
        cw_ref[3, rows0, :] = (
            partial_sub(c0_cw, 0, 0).astype(jnp.bfloat16))
        ccw_ref[3, rows0, :] = (
            partial_sub(c0_ccw, 1, 0).astype(jnp.bfloat16))

        barrier_sem = pltpu.get_barrier_semaphore()
        for nbr in (left, right):
            pl.semaphore_signal(
                barrier_sem, inc=1,
                device_id=(nbr,), device_id_type=pl.DeviceIdType.MESH,
            )
        pl.semaphore_wait(barrier_sem, 2)

        fl = [start_sub(0, 0)]
        for s in range(1, N_SUB):
            rows = pl.ds(s * SUB, SUB)
            cw_ref[3, rows, :] = (
                partial_sub(c0_cw, 0, s).astype(jnp.bfloat16))
            ccw_ref[3, rows, :] = (
                partial_sub(c0_ccw, 1, s).astype(jnp.bfloat16))
            fl.append(start_sub(0, s))

        scale = sx_ref[0] * sw_ref[0]

        for h in range(N_DEV - 1):
            a_cw = [partial_sub(rc_cw(h), 0, s) for s in range(N_SUB)]
            a_ccw = [partial_sub(rc_ccw(h), 1, s) for s in range(N_SUB)]

            nxt = []
            for s in range(N_SUB):
                r_cw, r_ccw = fl[s]
                r_cw.wait()
                r_ccw.wait()
                rows = pl.ds(s * SUB, SUB)
                if h < N_DEV - 2:
                    cw_ref[h, rows, :] = (
                        cw_ref[h, rows, :].astype(jnp.float32)
                        + a_cw[s].astype(jnp.float32)
                    ).astype(jnp.bfloat16)
                    ccw_ref[h, rows, :] = (
                        ccw_ref[h, rows, :].astype(jnp.float32)
                        + a_ccw[s].astype(jnp.float32)
                    ).astype(jnp.bfloat16)
                    nxt.append(start_sub(h + 1, s))
                else:
                    acc_l = (cw_ref[h, rows, :].astype(jnp.float32)
                             + a_cw[s].astype(jnp.float32))
                    acc_r = (ccw_ref[h, rows, :].astype(jnp.float32)
                             + a_ccw[s].astype(jnp.float32))
                    out_ref[rows, pl.ds(0, N_HALF)] = acc_l * scale
                    out_ref[rows, pl.ds(N_HALF, N_HALF)] = acc_r * scale

            fl = nxt

        @functools.partial(
            pl.run_scoped, second_barrier=pltpu.SemaphoreType.REGULAR
        )
        def _(second_barrier):
            for nbr in (left, right):
                pl.semaphore_signal(
                    second_barrier, inc=1,
                    device_id=(nbr,), device_id_type=pl.DeviceIdType.MESH,
                )
            pl.semaphore_wait(second_barrier, 2)

    return pl.pallas_call(
        body,
        out_shape=jax.ShapeDtypeStruct((M_CH, N_OUT), jnp.float32),
        in_specs=[
            pl.BlockSpec(memory_space=pltpu.VMEM),
            pl.BlockSpec(memory_space=pltpu.VMEM),
            pl.BlockSpec(memory_space=pltpu.SMEM),
            pl.BlockSpec(memory_space=pltpu.SMEM),
        ],
        out_specs=pl.BlockSpec(memory_space=pltpu.VMEM),
        scratch_shapes=[
            pltpu.VMEM((N_DEV, M_CH, N_HALF), jnp.bfloat16),
            pltpu.VMEM((N_DEV, M_CH, N_HALF), jnp.bfloat16),
            pltpu.SemaphoreType.DMA((N_DEV - 1, N_SUB)),
            pltpu.SemaphoreType.DMA((N_DEV - 1, N_SUB)),
            pltpu.SemaphoreType.DMA((N_DEV - 1, N_SUB)),
            pltpu.SemaphoreType.DMA((N_DEV - 1, N_SUB)),
        ],
        compiler_params=pltpu.CompilerParams(
            collective_id=0,
            vmem_limit_bytes=100 * 1024 * 1024,
        ),
    )(x, w_mat, scale_x, scale_w)
